# baseline (speedup 1.0000x reference)
import functools

import jax
import jax.numpy as jnp
from jax import lax
from jax.experimental import pallas as pl
from jax.experimental.pallas import tpu as pltpu
from jax.experimental.pallas import tpu_sc as plsc

B = 16384
S = 10
F = 32
NC = 2
NS = 16
NW = NC * NS
BPW = B // NW
CB = 64
NCHUNK = BPW // CB
CN = CB * S
GB = 8
GN = GB * S
NVEC = GN // 16
NBLK = CB // GB

_mesh = plsc.VectorSubcoreMesh(core_axis_name="c", subcore_axis_name="s")


def _hsum_all_lanes(p, lane):
    for sft in (8, 4, 2, 1):
        p = p + jnp.take_along_axis(p, jnp.bitwise_xor(lane, sft), axis=0)
    return p


@functools.partial(
    pl.kernel,
    mesh=_mesh,
    compiler_params=pltpu.CompilerParams(needs_layout_passes=False),
    out_type=jax.ShapeDtypeStruct((B * S,), jnp.float32),
    scratch_types=[
        pltpu.VMEM((CN,), jnp.int32),
        pltpu.VMEM((CN,), jnp.int32),
        pltpu.VMEM((CB + 16,), jnp.int32),
        pltpu.VMEM((CB + 16,), jnp.int32),
        pltpu.VMEM((CN, 128), jnp.float32),
        pltpu.VMEM((CB + 16, 128), jnp.float32),
        pltpu.VMEM((CN,), jnp.float32),
        pltpu.SemaphoreType.DMA,
    ],
)
def _score_kernel(drow_hbm, doff_hbm, urow_hbm, uoff_hbm, doc_hbm, uemb_hbm,
                  out_hbm, drow_v, doff_v, urow_v, uoff_v, doc_v, usr_v,
                  sc_v, sem):
    wid = lax.axis_index("c") * NS + lax.axis_index("s")
    lane = lax.iota(jnp.int32, 16)
    zeros16i = jnp.zeros((16,), jnp.int32)

    def chunk_body(chunk, carry):
        nbase = wid * BPW * S + chunk * CN
        bbase = wid * BPW + chunk * CB
        pltpu.sync_copy(drow_hbm.at[pl.ds(nbase, CN)], drow_v)
        pltpu.sync_copy(doff_hbm.at[pl.ds(nbase, CN)], doff_v)
        pltpu.sync_copy(urow_hbm.at[pl.ds(bbase, CB)], urow_v.at[pl.ds(0, CB)])
        pltpu.sync_copy(uoff_hbm.at[pl.ds(bbase, CB)], uoff_v.at[pl.ds(0, CB)])
        urow_v[pl.ds(CB, 16)] = zeros16i
        cp_doc = pltpu.async_copy(doc_hbm.at[drow_v], doc_v, sem)
        cp_usr = pltpu.async_copy(uemb_hbm.at[urow_v], usr_v, sem)
        cp_doc.wait()
        cp_usr.wait()

        def block_body(blk, bcarry):
            base_b = blk * GB
            base_n = blk * GN
            uoff_vec = uoff_v[pl.ds(base_b, 16)]
            accs = [jnp.zeros((16,), jnp.float32)] * NVEC
            doff_vecs = [doff_v[pl.ds(base_n + v * 16, 16)]
                         for v in range(NVEC)]
            for i2 in range(GB):
                ub = jnp.take_along_axis(uoff_vec, jnp.full((16,), i2,
                                                            jnp.int32), axis=0)
                bib = jnp.broadcast_to(base_b + i2, (16,)).astype(jnp.int32)
                u0 = plsc.load_gather(usr_v, [bib, ub + lane])
                u1 = plsc.load_gather(usr_v, [bib, ub + lane + 16])
                for s in range(S):
                    n2 = i2 * S + s
                    v, ln = divmod(n2, 16)
                    ob = jnp.take_along_axis(
                        doff_vecs[v], jnp.full((16,), ln, jnp.int32), axis=0)
                    nb = jnp.broadcast_to(base_n + n2, (16,)).astype(jnp.int32)
                    d0 = plsc.load_gather(doc_v, [nb, ob + lane])
                    d1 = plsc.load_gather(doc_v, [nb, ob + lane + 16])
                    tot = _hsum_all_lanes(d0 * u0 + d1 * u1, lane)
                    accs[v] = jnp.where(lane == ln, tot, accs[v])
            for v in range(NVEC):
                sc_v[pl.ds(base_n + v * 16, 16)] = accs[v]
            return bcarry

        lax.fori_loop(0, NBLK, block_body, 0)
        pltpu.sync_copy(sc_v, out_hbm.at[pl.ds(nbase, CN)])
        return carry

    lax.fori_loop(0, NCHUNK, chunk_body, 0)


def kernel(item_ids, user_ids, docEmbed, userEmbed):
    flat_items = item_ids.reshape(-1).astype(jnp.int32)
    uids = user_ids.astype(jnp.int32)
    drow = flat_items >> 2
    doff = (flat_items & 3) << 5
    urow = uids >> 2
    uoff = (uids & 3) << 5
    doc128 = docEmbed.reshape(-1, 128)
    uemb128 = userEmbed.reshape(-1, 128)
    out = _score_kernel(drow, doff, urow, uoff, doc128, uemb128)
    return out.reshape(B, S)


# speedup vs baseline: 1.3244x; 1.3244x over previous
import functools

import jax
import jax.numpy as jnp
from jax import lax
from jax.experimental import pallas as pl
from jax.experimental.pallas import tpu as pltpu
from jax.experimental.pallas import tpu_sc as plsc

B = 16384
S = 10
F = 32
NC = 2
NS = 16
NW = NC * NS
BPW = B // NW
CB = 256
NCHUNK = BPW // CB
CN = CB * S
GB = 8
GN = GB * S
NVEC = GN // 16

_mesh = plsc.VectorSubcoreMesh(core_axis_name="c", subcore_axis_name="s")


def _hsum_all_lanes(p, lane):
    for sft in (8, 4, 2, 1):
        p = p + jnp.take_along_axis(p, jnp.bitwise_xor(lane, sft), axis=0)
    return p


@functools.partial(
    pl.kernel,
    mesh=_mesh,
    compiler_params=pltpu.CompilerParams(use_tc_tiling_on_sc=False,
                                         needs_layout_passes=False),
    out_type=jax.ShapeDtypeStruct((S, B), jnp.float32),
    scratch_types=[
        pltpu.VMEM((CN,), jnp.int32),
        pltpu.VMEM((CB,), jnp.int32),
        pltpu.VMEM((CN, F), jnp.float32),
        pltpu.VMEM((CB, F), jnp.float32),
        pltpu.VMEM((CN,), jnp.float32),
        pltpu.SemaphoreType.DMA,
    ],
)
def _score_kernel(items_hbm, user_hbm, doc_hbm, uemb_hbm, out_hbm,
                  iidx_v, uidx_v, doc_v, usr_v, sc_v, sem):
    wid = lax.axis_index("c") * NS + lax.axis_index("s")
    lane = lax.iota(jnp.int32, 16)

    def chunk_body(chunk, carry):
        bbase = wid * BPW + chunk * CB
        for s in range(S):
            pltpu.sync_copy(items_hbm.at[s, pl.ds(bbase, CB)],
                            iidx_v.at[pl.ds(s * CB, CB)])
        pltpu.sync_copy(user_hbm.at[pl.ds(bbase, CB)], uidx_v)
        cp_doc = pltpu.async_copy(doc_hbm.at[iidx_v], doc_v, sem)
        cp_usr = pltpu.async_copy(uemb_hbm.at[uidx_v], usr_v, sem)
        cp_doc.wait()
        cp_usr.wait()

        def block_body(g, bcarry):
            base_b = (g % (CB // 16)) * 16
            base_n = (g // (CB // 16)) * CB + base_b
            acc = jnp.zeros((16,), jnp.float32)
            for l in range(16):
                u0 = usr_v[base_b + l, pl.ds(0, 16)]
                u1 = usr_v[base_b + l, pl.ds(16, 16)]
                d0 = doc_v[base_n + l, pl.ds(0, 16)]
                d1 = doc_v[base_n + l, pl.ds(16, 16)]
                tot = _hsum_all_lanes(d0 * u0 + d1 * u1, lane)
                acc = jnp.where(lane == l, tot, acc)
            sc_v[pl.ds(base_n, 16)] = acc
            return bcarry

        lax.fori_loop(0, CN // 16, block_body, 0)
        for s in range(S):
            pltpu.sync_copy(sc_v.at[pl.ds(s * CB, CB)],
                            out_hbm.at[s, pl.ds(bbase, CB)])
        return carry

    lax.fori_loop(0, NCHUNK, chunk_body, 0)


def kernel(item_ids, user_ids, docEmbed, userEmbed):
    items_t = item_ids.T.astype(jnp.int32)
    uids = user_ids.astype(jnp.int32)
    out_t = _score_kernel(items_t, uids, docEmbed, userEmbed)
    return out_t.T
